# baseline (speedup 1.0000x reference)
import functools

import jax
import jax.numpy as jnp
from jax import lax
from jax.experimental import pallas as pl
from jax.experimental.pallas import tpu as pltpu
from jax.experimental.pallas import tpu_sc as plsc

M = 4096
N = 16384
B = 64

MB = 2048
NCI = 2048

SC_CORES = 2
SC_SUBCORES = 16
NW = SC_CORES * SC_SUBCORES
BPW = M // NW


def _argmin_body(aT_ref, b_ref, idx_ref):
    a0 = aT_ref[:, 0:1]
    a1 = aT_ref[:, 1:2]
    diag1 = a0 * a0 + a1 * a1
    aTb2 = (aT_ref[:, :] * 2.0).astype(jnp.bfloat16)
    lane = lax.broadcasted_iota(jnp.int32, (1, 128), 1)

    def combine(left, right):
        lv, li = left
        nv, ni = right
        keep = lv <= nv
        return jnp.where(keep, lv, nv), jnp.where(keep, li, ni)

    run = None
    for c in range(N // NCI):
        bc = b_ref[:, c * NCI:(c + 1) * NCI]
        b0 = bc[0:1, :]
        b1 = bc[1:2, :]
        diag2 = b0 * b0 + b1 * b1
        r2 = lax.dot_general(aTb2, bc.astype(jnp.bfloat16),
                             (((1,), (0,)), ((), ())),
                             preferred_element_type=jnp.float32)
        s = (diag1 + diag2) - r2
        u = lax.bitcast_convert_type(s, jnp.int32) ^ jnp.int32(-(2**31))
        pairs = [(u[:, k * 128:(k + 1) * 128],
                  jnp.broadcast_to(lane + (c * NCI + k * 128), (MB, 128)))
                 for k in range(NCI // 128)]
        while len(pairs) > 1:
            pairs = [combine(pairs[i], pairs[i + 1])
                     for i in range(0, len(pairs), 2)]
        run = pairs[0] if run is None else combine(run, pairs[0])

    rv, ri = run
    cmin = jnp.min(rv, axis=1, keepdims=True)
    idx_ref[:, :] = jnp.min(
        jnp.where(rv == cmin, ri, jnp.int32(2**30)),
        axis=1, keepdims=True)


def _nearest_idx(aT, b):
    return pl.pallas_call(
        _argmin_body,
        grid=(M // MB,),
        in_specs=[
            pl.BlockSpec((MB, 2), lambda i: (i, 0)),
            pl.BlockSpec((2, N), lambda i: (0, 0)),
        ],
        out_specs=pl.BlockSpec((MB, 1), lambda i: (i, 0)),
        out_shape=jax.ShapeDtypeStruct((M, 1), jnp.int32),
    )(aT, b)


GF = B // NW


def _gather_body(vals_hbm, idx_hbm, out_hbm, idx_v, out_v):
    wid = lax.axis_index("s") * SC_CORES + lax.axis_index("c")
    pltpu.sync_copy(idx_hbm, idx_v)
    for f in range(GF):
        row = wid * GF + f
        pltpu.sync_copy(vals_hbm.at[pl.ds(row * N, N)].at[idx_v], out_v)
        pltpu.sync_copy(out_v, out_hbm.at[pl.ds(row * M, M)])


@functools.cache
def _sc_gather():
    return pl.kernel(
        _gather_body,
        out_type=jax.ShapeDtypeStruct((B * M,), jnp.float32),
        mesh=plsc.VectorSubcoreMesh(
            core_axis_name="c", subcore_axis_name="s",
            num_cores=SC_CORES, num_subcores=SC_SUBCORES),
        scratch_types=[
            pltpu.VMEM((M,), jnp.int32),
            pltpu.VMEM((M,), jnp.float32),
        ],
    )


def kernel(interp_points, values_points, values):
    aT = interp_points.T
    idx = _nearest_idx(aT, values_points)
    flat = _sc_gather()(values.reshape(B * N), idx.reshape(M))
    return flat.reshape(B, M)[:, :, None]


# speedup vs baseline: 1.2045x; 1.2045x over previous
import functools

import jax
import jax.numpy as jnp
from jax import lax
from jax.experimental import pallas as pl
from jax.experimental.pallas import tpu as pltpu
from jax.experimental.pallas import tpu_sc as plsc

M = 4096
N = 16384
B = 64

MB = 1024
NCI = 4096

SC_CORES = 2
SC_SUBCORES = 16
NW = SC_CORES * SC_SUBCORES
BPW = M // NW


def _argmin_body(aT_ref, b_ref, idx_ref):
    a0 = aT_ref[:, 0:1]
    a1 = aT_ref[:, 1:2]
    diag1 = a0 * a0 + a1 * a1
    aTb2 = (aT_ref[:, :] * 2.0).astype(jnp.bfloat16)
    lane = lax.broadcasted_iota(jnp.int32, (1, 128), 1)

    def combine(left, right):
        lv, li = left
        nv, ni = right
        keep = lv <= nv
        return jnp.where(keep, lv, nv), jnp.where(keep, li, ni)

    run = None
    for c in range(N // NCI):
        bc = b_ref[:, c * NCI:(c + 1) * NCI]
        b0 = bc[0:1, :]
        b1 = bc[1:2, :]
        diag2 = b0 * b0 + b1 * b1
        r2 = lax.dot_general(aTb2, bc.astype(jnp.bfloat16),
                             (((1,), (0,)), ((), ())),
                             preferred_element_type=jnp.float32)
        s = (diag1 + diag2) - r2
        u = lax.bitcast_convert_type(s, jnp.int32) ^ jnp.int32(-(2**31))
        pairs = [(u[:, k * 128:(k + 1) * 128],
                  jnp.broadcast_to(lane + (c * NCI + k * 128), (MB, 128)))
                 for k in range(NCI // 128)]
        while len(pairs) > 1:
            pairs = [combine(pairs[i], pairs[i + 1])
                     for i in range(0, len(pairs), 2)]
        run = pairs[0] if run is None else combine(run, pairs[0])

    rv, ri = run
    cmin = jnp.min(rv, axis=1, keepdims=True)
    idx_ref[:, :] = jnp.min(
        jnp.where(rv == cmin, ri, jnp.int32(2**30)),
        axis=1, keepdims=True)


def _nearest_idx(aT, b):
    return pl.pallas_call(
        _argmin_body,
        grid=(M // MB,),
        in_specs=[
            pl.BlockSpec((MB, 2), lambda i: (i, 0)),
            pl.BlockSpec((2, N), lambda i: (0, 0)),
        ],
        out_specs=pl.BlockSpec((MB, 1), lambda i: (i, 0)),
        out_shape=jax.ShapeDtypeStruct((M, 1), jnp.int32),
    )(aT, b)


GF = B // NW


def _gather_body(vals_hbm, idx_hbm, out_hbm, idx_v, out_v):
    wid = lax.axis_index("s") * SC_CORES + lax.axis_index("c")
    pltpu.sync_copy(idx_hbm, idx_v)
    for f in range(GF):
        row = wid * GF + f
        pltpu.sync_copy(vals_hbm.at[pl.ds(row * N, N)].at[idx_v], out_v)
        pltpu.sync_copy(out_v, out_hbm.at[pl.ds(row * M, M)])


@functools.cache
def _sc_gather():
    return pl.kernel(
        _gather_body,
        out_type=jax.ShapeDtypeStruct((B * M,), jnp.float32),
        mesh=plsc.VectorSubcoreMesh(
            core_axis_name="c", subcore_axis_name="s",
            num_cores=SC_CORES, num_subcores=SC_SUBCORES),
        scratch_types=[
            pltpu.VMEM((M,), jnp.int32),
            pltpu.VMEM((M,), jnp.float32),
        ],
    )


def kernel(interp_points, values_points, values):
    aT = interp_points.T
    idx = _nearest_idx(aT, values_points)
    flat = _sc_gather()(values.reshape(B * N), idx.reshape(M))
    return flat.reshape(B, M)[:, :, None]


# speedup vs baseline: 1.2103x; 1.0048x over previous
import functools

import jax
import jax.numpy as jnp
from jax import lax
from jax.experimental import pallas as pl
from jax.experimental.pallas import tpu as pltpu
from jax.experimental.pallas import tpu_sc as plsc

M = 4096
N = 16384
B = 64

MB = 1024
NCI = 2048

SC_CORES = 2
SC_SUBCORES = 16
NW = SC_CORES * SC_SUBCORES
BPW = M // NW


def _argmin_body(aT_ref, b_ref, idx_ref):
    a0 = aT_ref[:, 0:1]
    a1 = aT_ref[:, 1:2]
    diag1 = a0 * a0 + a1 * a1
    aTb2 = (aT_ref[:, :] * 2.0).astype(jnp.bfloat16)
    lane = lax.broadcasted_iota(jnp.int32, (1, 128), 1)

    def combine(left, right):
        lv, li = left
        nv, ni = right
        keep = lv <= nv
        return jnp.where(keep, lv, nv), jnp.where(keep, li, ni)

    run = None
    for c in range(N // NCI):
        bc = b_ref[:, c * NCI:(c + 1) * NCI]
        b0 = bc[0:1, :]
        b1 = bc[1:2, :]
        diag2 = b0 * b0 + b1 * b1
        r2 = lax.dot_general(aTb2, bc.astype(jnp.bfloat16),
                             (((1,), (0,)), ((), ())),
                             preferred_element_type=jnp.float32)
        s = (diag1 + diag2) - r2
        u = lax.bitcast_convert_type(s, jnp.int32) ^ jnp.int32(-(2**31))
        pairs = [(u[:, k * 128:(k + 1) * 128],
                  jnp.broadcast_to(lane + (c * NCI + k * 128), (MB, 128)))
                 for k in range(NCI // 128)]
        while len(pairs) > 1:
            pairs = [combine(pairs[i], pairs[i + 1])
                     for i in range(0, len(pairs), 2)]
        run = pairs[0] if run is None else combine(run, pairs[0])

    rv, ri = run
    cmin = jnp.min(rv, axis=1, keepdims=True)
    idx_ref[:, :] = jnp.min(
        jnp.where(rv == cmin, ri, jnp.int32(2**30)),
        axis=1, keepdims=True)


def _nearest_idx(aT, b):
    return pl.pallas_call(
        _argmin_body,
        grid=(M // MB,),
        in_specs=[
            pl.BlockSpec((MB, 2), lambda i: (i, 0)),
            pl.BlockSpec((2, N), lambda i: (0, 0)),
        ],
        out_specs=pl.BlockSpec((MB, 1), lambda i: (i, 0)),
        out_shape=jax.ShapeDtypeStruct((M, 1), jnp.int32),
    )(aT, b)


GF = B // NW


def _gather_body(vals_hbm, idx_hbm, out_hbm, idx_v, out_v):
    wid = lax.axis_index("s") * SC_CORES + lax.axis_index("c")
    pltpu.sync_copy(idx_hbm, idx_v)
    for f in range(GF):
        row = wid * GF + f
        pltpu.sync_copy(vals_hbm.at[pl.ds(row * N, N)].at[idx_v], out_v)
        pltpu.sync_copy(out_v, out_hbm.at[pl.ds(row * M, M)])


@functools.cache
def _sc_gather():
    return pl.kernel(
        _gather_body,
        out_type=jax.ShapeDtypeStruct((B * M,), jnp.float32),
        mesh=plsc.VectorSubcoreMesh(
            core_axis_name="c", subcore_axis_name="s",
            num_cores=SC_CORES, num_subcores=SC_SUBCORES),
        scratch_types=[
            pltpu.VMEM((M,), jnp.int32),
            pltpu.VMEM((M,), jnp.float32),
        ],
    )


def kernel(interp_points, values_points, values):
    aT = interp_points.T
    idx = _nearest_idx(aT, values_points)
    flat = _sc_gather()(values.reshape(B * N), idx.reshape(M))
    return flat.reshape(B, M)[:, :, None]


# speedup vs baseline: 1.2129x; 1.0021x over previous
import functools

import jax
import jax.numpy as jnp
from jax import lax
from jax.experimental import pallas as pl
from jax.experimental.pallas import tpu as pltpu
from jax.experimental.pallas import tpu_sc as plsc

M = 4096
N = 16384
B = 64

MB = 1024
NCI = 1024

SC_CORES = 2
SC_SUBCORES = 16
NW = SC_CORES * SC_SUBCORES
BPW = M // NW


def _argmin_body(aT_ref, b_ref, idx_ref):
    a0 = aT_ref[:, 0:1]
    a1 = aT_ref[:, 1:2]
    diag1 = a0 * a0 + a1 * a1
    aTb2 = (aT_ref[:, :] * 2.0).astype(jnp.bfloat16)
    lane = lax.broadcasted_iota(jnp.int32, (1, 128), 1)

    def combine(left, right):
        lv, li = left
        nv, ni = right
        keep = lv <= nv
        return jnp.where(keep, lv, nv), jnp.where(keep, li, ni)

    run = None
    for c in range(N // NCI):
        bc = b_ref[:, c * NCI:(c + 1) * NCI]
        b0 = bc[0:1, :]
        b1 = bc[1:2, :]
        diag2 = b0 * b0 + b1 * b1
        r2 = lax.dot_general(aTb2, bc.astype(jnp.bfloat16),
                             (((1,), (0,)), ((), ())),
                             preferred_element_type=jnp.float32)
        s = (diag1 + diag2) - r2
        u = lax.bitcast_convert_type(s, jnp.int32) ^ jnp.int32(-(2**31))
        pairs = [(u[:, k * 128:(k + 1) * 128],
                  jnp.broadcast_to(lane + (c * NCI + k * 128), (MB, 128)))
                 for k in range(NCI // 128)]
        while len(pairs) > 1:
            pairs = [combine(pairs[i], pairs[i + 1])
                     for i in range(0, len(pairs), 2)]
        run = pairs[0] if run is None else combine(run, pairs[0])

    rv, ri = run
    cmin = jnp.min(rv, axis=1, keepdims=True)
    idx_ref[:, :] = jnp.min(
        jnp.where(rv == cmin, ri, jnp.int32(2**30)),
        axis=1, keepdims=True)


def _nearest_idx(aT, b):
    return pl.pallas_call(
        _argmin_body,
        grid=(M // MB,),
        in_specs=[
            pl.BlockSpec((MB, 2), lambda i: (i, 0)),
            pl.BlockSpec((2, N), lambda i: (0, 0)),
        ],
        out_specs=pl.BlockSpec((MB, 1), lambda i: (i, 0)),
        out_shape=jax.ShapeDtypeStruct((M, 1), jnp.int32),
    )(aT, b)


GF = B // NW


def _gather_body(vals_hbm, idx_hbm, out_hbm, idx_v, out_v):
    wid = lax.axis_index("s") * SC_CORES + lax.axis_index("c")
    pltpu.sync_copy(idx_hbm, idx_v)
    for f in range(GF):
        row = wid * GF + f
        pltpu.sync_copy(vals_hbm.at[pl.ds(row * N, N)].at[idx_v], out_v)
        pltpu.sync_copy(out_v, out_hbm.at[pl.ds(row * M, M)])


@functools.cache
def _sc_gather():
    return pl.kernel(
        _gather_body,
        out_type=jax.ShapeDtypeStruct((B * M,), jnp.float32),
        mesh=plsc.VectorSubcoreMesh(
            core_axis_name="c", subcore_axis_name="s",
            num_cores=SC_CORES, num_subcores=SC_SUBCORES),
        scratch_types=[
            pltpu.VMEM((M,), jnp.int32),
            pltpu.VMEM((M,), jnp.float32),
        ],
    )


def kernel(interp_points, values_points, values):
    aT = interp_points.T
    idx = _nearest_idx(aT, values_points)
    flat = _sc_gather()(values.reshape(B * N), idx.reshape(M))
    return flat.reshape(B, M)[:, :, None]
